# Initial kernel scaffold; baseline (speedup 1.0000x reference)
import jax
import jax.numpy as jnp
from jax.experimental import pallas as pl


def kernel(h, edge_index, edge_weight):
    raise NotImplementedError("write your pallas kernel here")


# speedup vs baseline: 4.4574x; 4.4574x over previous
import functools

import jax
import jax.numpy as jnp
from jax import lax
from jax.experimental import pallas as pl
from jax.experimental.pallas import tpu as pltpu
from jax.experimental.pallas import tpu_sc as plsc

N_NODES = 10000
N_EDGES = 320000
D_FEAT = 128

NC = 2
NS = 16
NW = NC * NS
EDGES_PER_WORKER = N_EDGES // NW
CHUNK = 80
NUM_CHUNKS = EDGES_PER_WORKER // CHUNK
PAD_NODES = 10240
ROWS_PER_TILE = PAD_NODES // NS
ZROWS = 128
LANES = 16


def _sc_body(h_hbm, src_hbm, dst_hbm, w_hbm, out_hbm,
             src_v, dst_v, w_v, rows_v, stage_v, acc_sh, sem):
    c = lax.axis_index("c")
    s = lax.axis_index("s")
    wid = c * NS + s
    base = wid * EDGES_PER_WORKER

    @pl.loop(0, ZROWS)
    def _zero_stage(i):
        for k in range(D_FEAT // LANES):
            stage_v[i, pl.ds(k * LANES, LANES)] = jnp.zeros((LANES,), jnp.float32)

    @pl.loop(0, ROWS_PER_TILE // ZROWS)
    def _zero_acc(j):
        row0 = s * ROWS_PER_TILE + j * ZROWS
        pltpu.sync_copy(stage_v, acc_sh.at[pl.ds(row0, ZROWS)])

    plsc.subcore_barrier()

    @pl.loop(0, NUM_CHUNKS)
    def _chunk(n):
        off = base + n * CHUNK
        pltpu.sync_copy(src_hbm.at[pl.ds(off, CHUNK)], src_v)
        pltpu.sync_copy(dst_hbm.at[pl.ds(off, CHUNK)], dst_v)
        pltpu.sync_copy(w_hbm.at[pl.ds(off, CHUNK)], w_v)
        pltpu.async_copy(h_hbm.at[src_v], rows_v, sem).wait()

        @pl.loop(0, CHUNK // LANES)
        def _scale(g):
            wv = w_v[pl.ds(g * LANES, LANES)]
            for j in range(LANES):
                w = wv[j]
                i = g * LANES + j
                for k in range(D_FEAT // LANES):
                    sl = pl.ds(k * LANES, LANES)
                    rows_v[i, sl] = rows_v[i, sl] * w

        pltpu.sync_copy(rows_v, acc_sh.at[dst_v], add=True)

    plsc.subcore_barrier()

    @pl.loop(0, ROWS_PER_TILE // ZROWS)
    def _writeout(j):
        row0 = s * ROWS_PER_TILE + j * ZROWS
        pltpu.sync_copy(acc_sh.at[pl.ds(row0, ZROWS)], stage_v)
        pltpu.sync_copy(stage_v, out_hbm.at[c, pl.ds(row0, ZROWS)])


_sc_call = functools.partial(
    pl.kernel,
    out_type=jax.ShapeDtypeStruct((NC, PAD_NODES, D_FEAT), jnp.float32),
    mesh=plsc.VectorSubcoreMesh(core_axis_name="c", subcore_axis_name="s"),
    scratch_types=[
        pltpu.VMEM((CHUNK,), jnp.int32),
        pltpu.VMEM((CHUNK,), jnp.int32),
        pltpu.VMEM((CHUNK,), jnp.float32),
        pltpu.VMEM((CHUNK, D_FEAT), jnp.float32),
        pltpu.VMEM((ZROWS, D_FEAT), jnp.float32),
        pltpu.VMEM_SHARED((PAD_NODES, D_FEAT), jnp.float32),
        pltpu.SemaphoreType.DMA,
    ],
)(_sc_body)


def _merge_body(p_ref, o_ref):
    o_ref[...] = p_ref[0] + p_ref[1]


MERGE_BLK = 2000

_merge = pl.pallas_call(
    _merge_body,
    grid=(N_NODES // MERGE_BLK,),
    in_specs=[pl.BlockSpec((NC, MERGE_BLK, D_FEAT), lambda i: (0, i, 0))],
    out_specs=pl.BlockSpec((MERGE_BLK, D_FEAT), lambda i: (i, 0)),
    out_shape=jax.ShapeDtypeStruct((N_NODES, D_FEAT), jnp.float32),
)


@jax.jit
def kernel(h, edge_index, edge_weight):
    src = edge_index[0].astype(jnp.int32)
    dst = edge_index[1].astype(jnp.int32)
    w = edge_weight.reshape(-1)
    partials = _sc_call(h, src, dst, w)
    return _merge(partials)


# speedup vs baseline: 10.6018x; 2.3785x over previous
import dataclasses
import functools

import jax
import jax.numpy as jnp
from jax import lax
from jax.experimental import pallas as pl
from jax.experimental.pallas import tpu as pltpu
from jax.experimental.pallas import tpu_sc as plsc

N_NODES = 10000
N_EDGES = 320000
D_FEAT = 128

NC = 2
NS = 16
NW = NC * NS
EDGES_PER_WORKER = N_EDGES // NW
CHUNK = 80
NUM_CHUNKS = EDGES_PER_WORKER // CHUNK
PAD_NODES = 10240
ROWS_PER_TILE = PAD_NODES // NS
ZROWS = CHUNK
LANES = 16
DEPTH = 4


def _sc_body(h_hbm, idx_hbm, w_hbm, out_hbm, idx_v, w_v, rows_v, acc_sh,
             *sems):
    isem = sems[0:DEPTH]
    gsem = sems[DEPTH:2 * DEPTH]
    ssem = sems[2 * DEPTH:3 * DEPTH]

    c = lax.axis_index("c")
    s = lax.axis_index("s")
    wid = c * NS + s

    def issue_idx(n, j):
        pltpu.async_copy(idx_hbm.at[wid, n], idx_v.at[j], isem[j])
        pltpu.async_copy(w_hbm.at[wid, n], w_v.at[j], isem[j])

    def wait_idx(j):
        pltpu.make_async_copy(idx_hbm.at[wid, 0], idx_v.at[j], isem[j]).wait()
        pltpu.make_async_copy(w_hbm.at[wid, 0], w_v.at[j], isem[j]).wait()

    def issue_gather(j):
        pltpu.async_copy(h_hbm.at[idx_v.at[j, 0]], rows_v.at[j], gsem[j])

    def wait_gather(j):
        pltpu.make_async_copy(h_hbm.at[idx_v.at[j, 0]], rows_v.at[j],
                              gsem[j]).wait()

    def issue_scatter(j):
        pltpu.async_copy(rows_v.at[j], acc_sh.at[idx_v.at[j, 1]], ssem[j],
                         add=True)

    def wait_scatter(j):
        pltpu.make_async_copy(rows_v.at[j], acc_sh.at[idx_v.at[j, 1]],
                              ssem[j]).wait()

    @pl.loop(0, ZROWS)
    def _zero_stage(i):
        for k in range(D_FEAT // LANES):
            rows_v[0, i, pl.ds(k * LANES, LANES)] = jnp.zeros((LANES,),
                                                              jnp.float32)

    @pl.loop(0, ROWS_PER_TILE // ZROWS)
    def _zero_acc(j):
        row0 = s * ROWS_PER_TILE + j * ZROWS
        pltpu.sync_copy(rows_v.at[0], acc_sh.at[pl.ds(row0, ZROWS)])

    plsc.subcore_barrier()

    issue_idx(0, 0)
    issue_idx(1, 1)
    wait_idx(0)
    issue_gather(0)

    @pl.loop(0, (NUM_CHUNKS + DEPTH - 1) // DEPTH)
    def _quad(m):
        for b in range(DEPTH):
            n = m * DEPTH + b

            @pl.when(n < NUM_CHUNKS)
            def _():
                @pl.when(n >= 2)
                def _():
                    wait_scatter((b + 2) % DEPTH)

                @pl.when(n + 2 < NUM_CHUNKS)
                def _():
                    issue_idx(n + 2, (b + 2) % DEPTH)

                @pl.when(n + 1 < NUM_CHUNKS)
                def _():
                    wait_idx((b + 1) % DEPTH)
                    issue_gather((b + 1) % DEPTH)

                wait_gather(b)

                @pl.loop(0, CHUNK // LANES)
                def _scale(g):
                    wv = w_v[b, pl.ds(g * LANES, LANES)]
                    for jj in range(LANES):
                        w = wv[jj]
                        i = g * LANES + jj
                        for k in range(D_FEAT // LANES):
                            sl = pl.ds(k * LANES, LANES)
                            rows_v[b, i, sl] = rows_v[b, i, sl] * w

                issue_scatter(b)

    wait_scatter((NUM_CHUNKS - 2) % DEPTH)
    wait_scatter((NUM_CHUNKS - 1) % DEPTH)

    plsc.subcore_barrier()

    @pl.loop(0, ROWS_PER_TILE // ZROWS)
    def _writeout(j):
        row0 = s * ROWS_PER_TILE + j * ZROWS
        pltpu.sync_copy(acc_sh.at[pl.ds(row0, ZROWS)], rows_v.at[0])
        pltpu.sync_copy(rows_v.at[0], out_hbm.at[c, pl.ds(row0, ZROWS)])


_cp = pltpu.CompilerParams()
if "needs_layout_passes" in pltpu.CompilerParams.__dataclass_fields__:
    _cp = dataclasses.replace(_cp, needs_layout_passes=False)

_sc_call = functools.partial(
    pl.kernel,
    compiler_params=_cp,
    out_type=jax.ShapeDtypeStruct((NC, PAD_NODES, D_FEAT), jnp.float32),
    mesh=plsc.VectorSubcoreMesh(core_axis_name="c", subcore_axis_name="s"),
    scratch_types=[
        pltpu.VMEM((DEPTH, 2, CHUNK), jnp.int32),
        pltpu.VMEM((DEPTH, 128), jnp.float32),
        pltpu.VMEM((DEPTH, CHUNK, D_FEAT), jnp.float32),
        pltpu.VMEM_SHARED((PAD_NODES, D_FEAT), jnp.float32),
    ] + [pltpu.SemaphoreType.DMA] * (3 * DEPTH),
)(_sc_body)


def _merge_body(p_ref, o_ref):
    o_ref[...] = p_ref[0] + p_ref[1]


MERGE_BLK = 2000

_merge = pl.pallas_call(
    _merge_body,
    grid=(N_NODES // MERGE_BLK,),
    in_specs=[pl.BlockSpec((NC, MERGE_BLK, D_FEAT), lambda i: (0, i, 0))],
    out_specs=pl.BlockSpec((MERGE_BLK, D_FEAT), lambda i: (i, 0)),
    out_shape=jax.ShapeDtypeStruct((N_NODES, D_FEAT), jnp.float32),
)


@jax.jit
def kernel(h, edge_index, edge_weight):
    src = edge_index[0].astype(jnp.int32).reshape(NW, NUM_CHUNKS, CHUNK)
    dst = edge_index[1].astype(jnp.int32).reshape(NW, NUM_CHUNKS, CHUNK)
    idx = jnp.stack([src, dst], axis=2)
    w = edge_weight.reshape(NW, NUM_CHUNKS, CHUNK)
    w = jnp.pad(w, ((0, 0), (0, 0), (0, 128 - CHUNK)))
    partials = _sc_call(h, idx, w)
    return _merge(partials)


# speedup vs baseline: 13.1647x; 1.2417x over previous
import dataclasses
import functools

import jax
import jax.numpy as jnp
from jax import lax
from jax.experimental import pallas as pl
from jax.experimental.pallas import tpu as pltpu
from jax.experimental.pallas import tpu_sc as plsc

N_NODES = 10000
N_EDGES = 320000
D_FEAT = 128

NC = 2
NS = 16
NW = NC * NS
CHUNK = 128
TOTAL_CHUNKS = N_EDGES // CHUNK
MAX_CHUNKS = -(-TOTAL_CHUNKS // NW)
LANES = 16
DEPTH = 3
WO_ROWS = 80


def _sc_body(h_hbm, ei_hbm, w_hbm, out_hbm, idx_v, w_v, rows_v, acc_sh,
             *sems):
    isem = sems[0:DEPTH]
    gsem = sems[DEPTH:2 * DEPTH]
    ssem = sems[2 * DEPTH:3 * DEPTH]

    c = lax.axis_index("c")
    s = lax.axis_index("s")
    wid = c * NS + s
    nchunks = jnp.where(wid < TOTAL_CHUNKS - (MAX_CHUNKS - 1) * NW,
                        MAX_CHUNKS, MAX_CHUNKS - 1)

    def issue_idx(n, j):
        off = (wid + n * NW) * CHUNK
        pltpu.async_copy(ei_hbm.at[:, pl.ds(off, CHUNK)],
                         idx_v.at[j], isem[j])
        pltpu.async_copy(w_hbm.at[0, pl.ds(off, CHUNK)], w_v.at[j], isem[j])

    def wait_idx(j):
        pltpu.make_async_copy(ei_hbm.at[:, pl.ds(0, CHUNK)], idx_v.at[j],
                              isem[j]).wait()
        pltpu.make_async_copy(w_hbm.at[0, pl.ds(0, CHUNK)], w_v.at[j],
                              isem[j]).wait()

    def issue_gather(j):
        pltpu.async_copy(h_hbm.at[idx_v.at[j, 0]], rows_v.at[j], gsem[j])

    def wait_gather(j):
        pltpu.make_async_copy(h_hbm.at[idx_v.at[j, 0]], rows_v.at[j],
                              gsem[j]).wait()

    def issue_scatter(j):
        pltpu.async_copy(rows_v.at[j], acc_sh.at[idx_v.at[j, 1]], ssem[j],
                         add=True)

    def wait_scatter(j):
        pltpu.make_async_copy(rows_v.at[j], acc_sh.at[idx_v.at[j, 1]],
                              ssem[j]).wait()

    row_base = s * 640
    wo_trips = jnp.where(s == NS - 1, 5, 8)

    @pl.loop(0, WO_ROWS)
    def _zero_stage(i):
        for k in range(D_FEAT // LANES):
            rows_v[0, i, pl.ds(k * LANES, LANES)] = jnp.zeros((LANES,),
                                                              jnp.float32)

    def _zero_acc(j, _):
        pltpu.sync_copy(rows_v.at[0, pl.ds(0, WO_ROWS)],
                        acc_sh.at[pl.ds(row_base + j * WO_ROWS, WO_ROWS)])
        return _

    lax.fori_loop(0, wo_trips, _zero_acc, None)

    plsc.subcore_barrier()

    issue_idx(0, 0)
    issue_idx(1, 1)
    wait_idx(0)
    issue_gather(0)

    @pl.loop(0, (MAX_CHUNKS + DEPTH - 1) // DEPTH)
    def _triple(m):
        for b in range(DEPTH):
            n = m * DEPTH + b

            @pl.when(n < nchunks)
            def _():
                @pl.when(n + 1 < nchunks)
                def _():
                    wait_idx((b + 1) % DEPTH)
                    issue_gather((b + 1) % DEPTH)

                wait_gather(b)

                @pl.loop(0, CHUNK // LANES)
                def _scale(g):
                    wv = w_v[b, pl.ds(g * LANES, LANES)]
                    for jj in range(LANES):
                        w = wv[jj]
                        i = g * LANES + jj
                        for k in range(D_FEAT // LANES):
                            sl = pl.ds(k * LANES, LANES)
                            rows_v[b, i, sl] = rows_v[b, i, sl] * w

                issue_scatter(b)

                @pl.when(n >= 1)
                def _():
                    wait_scatter((b + 2) % DEPTH)

                @pl.when(n + 2 < nchunks)
                def _():
                    issue_idx(n + 2, (b + 2) % DEPTH)

    @pl.when(nchunks == MAX_CHUNKS)
    def _():
        wait_scatter((MAX_CHUNKS - 1) % DEPTH)

    @pl.when(nchunks == MAX_CHUNKS - 1)
    def _():
        wait_scatter((MAX_CHUNKS - 2) % DEPTH)

    plsc.subcore_barrier()

    def _writeout(j, _):
        row0 = row_base + j * WO_ROWS
        pltpu.sync_copy(acc_sh.at[pl.ds(row0, WO_ROWS)],
                        rows_v.at[0, pl.ds(0, WO_ROWS)])
        pltpu.sync_copy(rows_v.at[0, pl.ds(0, WO_ROWS)],
                        out_hbm.at[c, pl.ds(row0, WO_ROWS)])
        return _

    lax.fori_loop(0, wo_trips, _writeout, None)


_cp = pltpu.CompilerParams()
if "needs_layout_passes" in pltpu.CompilerParams.__dataclass_fields__:
    _cp = dataclasses.replace(_cp, needs_layout_passes=False)

_sc_call = functools.partial(
    pl.kernel,
    compiler_params=_cp,
    out_type=jax.ShapeDtypeStruct((NC, N_NODES, D_FEAT), jnp.float32),
    mesh=plsc.VectorSubcoreMesh(core_axis_name="c", subcore_axis_name="s"),
    scratch_types=[
        pltpu.VMEM((DEPTH, 2, CHUNK), jnp.int32),
        pltpu.VMEM((DEPTH, CHUNK), jnp.float32),
        pltpu.VMEM((DEPTH, CHUNK, D_FEAT), jnp.float32),
        pltpu.VMEM_SHARED((N_NODES, D_FEAT), jnp.float32),
    ] + [pltpu.SemaphoreType.DMA] * (3 * DEPTH),
)(_sc_body)


def _merge_body(p_ref, o_ref):
    o_ref[...] = p_ref[0] + p_ref[1]


MERGE_BLK = 2000

_merge = pl.pallas_call(
    _merge_body,
    grid=(N_NODES // MERGE_BLK,),
    in_specs=[pl.BlockSpec((NC, MERGE_BLK, D_FEAT), lambda i: (0, i, 0))],
    out_specs=pl.BlockSpec((MERGE_BLK, D_FEAT), lambda i: (i, 0)),
    out_shape=jax.ShapeDtypeStruct((N_NODES, D_FEAT), jnp.float32),
)


@jax.jit
def kernel(h, edge_index, edge_weight):
    wr = edge_weight.reshape(1, N_EDGES)
    partials = _sc_call(h, edge_index.astype(jnp.int32), wr)
    return _merge(partials)


# speedup vs baseline: 13.2511x; 1.0066x over previous
import dataclasses
import functools

import jax
import jax.numpy as jnp
from jax import lax
from jax.experimental import pallas as pl
from jax.experimental.pallas import tpu as pltpu
from jax.experimental.pallas import tpu_sc as plsc

N_NODES = 10000
N_EDGES = 320000
D_FEAT = 128

NC = 2
NS = 16
NW = NC * NS
CHUNK = 128
TOTAL_CHUNKS = N_EDGES // CHUNK
MAX_CHUNKS = -(-TOTAL_CHUNKS // NW)
LANES = 16
DEPTH = 3
WO_ROWS = 80


def _sc_body(h_hbm, ei_hbm, w_hbm, out_hbm, idx_v, w_v, rows_v, acc_sh,
             *sems):
    isem = sems[0:DEPTH]
    gsem = sems[DEPTH:2 * DEPTH]
    ssem = sems[2 * DEPTH:3 * DEPTH]

    c = lax.axis_index("c")
    s = lax.axis_index("s")
    wid = c * NS + s
    nchunks = jnp.where(wid < TOTAL_CHUNKS - (MAX_CHUNKS - 1) * NW,
                        MAX_CHUNKS, MAX_CHUNKS - 1)

    def issue_idx(n, j):
        off = (wid + n * NW) * CHUNK
        pltpu.async_copy(ei_hbm.at[:, pl.ds(off, CHUNK)],
                         idx_v.at[j], isem[j])
        pltpu.async_copy(w_hbm.at[0, pl.ds(off, CHUNK)], w_v.at[j], isem[j])

    def wait_idx(j):
        pltpu.make_async_copy(ei_hbm.at[:, pl.ds(0, CHUNK)], idx_v.at[j],
                              isem[j]).wait()
        pltpu.make_async_copy(w_hbm.at[0, pl.ds(0, CHUNK)], w_v.at[j],
                              isem[j]).wait()

    def issue_gather(j):
        pltpu.async_copy(h_hbm.at[idx_v.at[j, 0]], rows_v.at[j], gsem[j])

    def wait_gather(j):
        pltpu.make_async_copy(h_hbm.at[idx_v.at[j, 0]], rows_v.at[j],
                              gsem[j]).wait()

    def issue_scatter(j):
        pltpu.async_copy(rows_v.at[j], acc_sh.at[idx_v.at[j, 1]], ssem[j],
                         add=True)

    def wait_scatter(j):
        pltpu.make_async_copy(rows_v.at[j], acc_sh.at[idx_v.at[j, 1]],
                              ssem[j]).wait()

    row_base = s * 640
    wo_trips = jnp.where(s == NS - 1, 5, 8)

    @pl.loop(0, WO_ROWS)
    def _zero_stage(i):
        for k in range(D_FEAT // LANES):
            rows_v[0, i, pl.ds(k * LANES, LANES)] = jnp.zeros((LANES,),
                                                              jnp.float32)

    def _zero_acc(j, _):
        pltpu.sync_copy(rows_v.at[0, pl.ds(0, WO_ROWS)],
                        acc_sh.at[pl.ds(row_base + j * WO_ROWS, WO_ROWS)])
        return _

    lax.fori_loop(0, wo_trips, _zero_acc, None)

    plsc.subcore_barrier()

    issue_idx(0, 0)
    issue_idx(1, 1)
    wait_idx(0)
    issue_gather(0)

    @pl.loop(0, (MAX_CHUNKS + DEPTH - 1) // DEPTH)
    def _triple(m):
        for b in range(DEPTH):
            n = m * DEPTH + b

            @pl.when(n < nchunks)
            def _():
                @pl.when(n + 1 < nchunks)
                def _():
                    wait_idx((b + 1) % DEPTH)
                    issue_gather((b + 1) % DEPTH)

                wait_gather(b)

                @pl.loop(0, CHUNK // LANES)
                def _scale(g):
                    wv = w_v[b, pl.ds(g * LANES, LANES)]
                    for jj in range(LANES):
                        w = wv[jj]
                        i = g * LANES + jj
                        for k in range(D_FEAT // LANES):
                            sl = pl.ds(k * LANES, LANES)
                            rows_v[b, i, sl] = rows_v[b, i, sl] * w

                issue_scatter(b)

                @pl.when(n >= 1)
                def _():
                    wait_scatter((b + 2) % DEPTH)

                @pl.when(n + 2 < nchunks)
                def _():
                    issue_idx(n + 2, (b + 2) % DEPTH)

    @pl.when(nchunks == MAX_CHUNKS)
    def _():
        wait_scatter((MAX_CHUNKS - 1) % DEPTH)

    @pl.when(nchunks == MAX_CHUNKS - 1)
    def _():
        wait_scatter((MAX_CHUNKS - 2) % DEPTH)

    plsc.subcore_barrier()

    def _writeout(j, _):
        row0 = row_base + j * WO_ROWS
        pltpu.sync_copy(acc_sh.at[pl.ds(row0, WO_ROWS)],
                        out_hbm.at[c, pl.ds(row0, WO_ROWS)])
        return _

    lax.fori_loop(0, wo_trips, _writeout, None)


_cp = pltpu.CompilerParams()
if "needs_layout_passes" in pltpu.CompilerParams.__dataclass_fields__:
    _cp = dataclasses.replace(_cp, needs_layout_passes=False)

_sc_call = functools.partial(
    pl.kernel,
    compiler_params=_cp,
    out_type=jax.ShapeDtypeStruct((NC, N_NODES, D_FEAT), jnp.float32),
    mesh=plsc.VectorSubcoreMesh(core_axis_name="c", subcore_axis_name="s"),
    scratch_types=[
        pltpu.VMEM((DEPTH, 2, CHUNK), jnp.int32),
        pltpu.VMEM((DEPTH, CHUNK), jnp.float32),
        pltpu.VMEM((DEPTH, CHUNK, D_FEAT), jnp.float32),
        pltpu.VMEM_SHARED((N_NODES, D_FEAT), jnp.float32),
    ] + [pltpu.SemaphoreType.DMA] * (3 * DEPTH),
)(_sc_body)


def _merge_body(p_ref, o_ref):
    o_ref[...] = p_ref[0] + p_ref[1]


MERGE_BLK = 2000

_merge = pl.pallas_call(
    _merge_body,
    grid=(N_NODES // MERGE_BLK,),
    in_specs=[pl.BlockSpec((NC, MERGE_BLK, D_FEAT), lambda i: (0, i, 0))],
    out_specs=pl.BlockSpec((MERGE_BLK, D_FEAT), lambda i: (i, 0)),
    out_shape=jax.ShapeDtypeStruct((N_NODES, D_FEAT), jnp.float32),
)


@jax.jit
def kernel(h, edge_index, edge_weight):
    wr = edge_weight.reshape(1, N_EDGES)
    partials = _sc_call(h, edge_index.astype(jnp.int32), wr)
    return _merge(partials)


# speedup vs baseline: 13.2858x; 1.0026x over previous
import dataclasses
import functools

import jax
import jax.numpy as jnp
from jax import lax
from jax.experimental import pallas as pl
from jax.experimental.pallas import tpu as pltpu
from jax.experimental.pallas import tpu_sc as plsc

N_NODES = 10000
N_EDGES = 320000
D_FEAT = 128

NC = 2
NS = 16
NW = NC * NS
CHUNK = 128
TOTAL_CHUNKS = N_EDGES // CHUNK
MAX_CHUNKS = -(-TOTAL_CHUNKS // NW)
LANES = 16
DEPTH = 3
WO_ROWS = 80


def _sc_body(h_hbm, ei_hbm, w_hbm, out_hbm, idx_v, w_v, rows_v, acc_sh,
             *sems):
    isem = sems[0:DEPTH]
    gsem = sems[DEPTH:3 * DEPTH]
    ssem = sems[3 * DEPTH:4 * DEPTH]

    c = lax.axis_index("c")
    s = lax.axis_index("s")
    wid = c * NS + s
    nchunks = jnp.where(wid < TOTAL_CHUNKS - (MAX_CHUNKS - 1) * NW,
                        MAX_CHUNKS, MAX_CHUNKS - 1)

    def issue_idx(n, j):
        off = (wid + n * NW) * CHUNK
        pltpu.async_copy(ei_hbm.at[:, pl.ds(off, CHUNK)],
                         idx_v.at[j], isem[j])
        pltpu.async_copy(w_hbm.at[0, pl.ds(off, CHUNK)], w_v.at[j], isem[j])

    def wait_idx(j):
        pltpu.make_async_copy(ei_hbm.at[:, pl.ds(0, CHUNK)], idx_v.at[j],
                              isem[j]).wait()
        pltpu.make_async_copy(w_hbm.at[0, pl.ds(0, CHUNK)], w_v.at[j],
                              isem[j]).wait()

    HALF = CHUNK // 2

    def issue_gather(j):
        for hh in range(2):
            pltpu.async_copy(
                h_hbm.at[idx_v.at[j, 0, pl.ds(hh * HALF, HALF)]],
                rows_v.at[j, pl.ds(hh * HALF, HALF)], gsem[2 * j + hh])

    def wait_gather_half(j, hh):
        pltpu.make_async_copy(
            h_hbm.at[idx_v.at[j, 0, pl.ds(hh * HALF, HALF)]],
            rows_v.at[j, pl.ds(hh * HALF, HALF)], gsem[2 * j + hh]).wait()

    def issue_scatter(j):
        pltpu.async_copy(rows_v.at[j], acc_sh.at[idx_v.at[j, 1]], ssem[j],
                         add=True)

    def wait_scatter(j):
        pltpu.make_async_copy(rows_v.at[j], acc_sh.at[idx_v.at[j, 1]],
                              ssem[j]).wait()

    row_base = s * 640
    wo_trips = jnp.where(s == NS - 1, 5, 8)

    @pl.loop(0, WO_ROWS)
    def _zero_stage(i):
        for k in range(D_FEAT // LANES):
            rows_v[0, i, pl.ds(k * LANES, LANES)] = jnp.zeros((LANES,),
                                                              jnp.float32)

    def _zero_acc(j, _):
        pltpu.sync_copy(rows_v.at[0, pl.ds(0, WO_ROWS)],
                        acc_sh.at[pl.ds(row_base + j * WO_ROWS, WO_ROWS)])
        return _

    lax.fori_loop(0, wo_trips, _zero_acc, None)

    plsc.subcore_barrier()

    issue_idx(0, 0)
    issue_idx(1, 1)
    wait_idx(0)
    issue_gather(0)

    @pl.loop(0, (MAX_CHUNKS + DEPTH - 1) // DEPTH)
    def _triple(m):
        for b in range(DEPTH):
            n = m * DEPTH + b

            @pl.when(n < nchunks)
            def _():
                @pl.when(n + 1 < nchunks)
                def _():
                    wait_idx((b + 1) % DEPTH)
                    issue_gather((b + 1) % DEPTH)

                for hh in range(2):
                    wait_gather_half(b, hh)

                    @pl.loop(hh * (HALF // LANES),
                             (hh + 1) * (HALF // LANES))
                    def _scale(g):
                        wv = w_v[b, pl.ds(g * LANES, LANES)]
                        for jj in range(LANES):
                            w = wv[jj]
                            i = g * LANES + jj
                            for k in range(D_FEAT // LANES):
                                sl = pl.ds(k * LANES, LANES)
                                rows_v[b, i, sl] = rows_v[b, i, sl] * w

                issue_scatter(b)

                @pl.when(n >= 1)
                def _():
                    wait_scatter((b + 2) % DEPTH)

                @pl.when(n + 2 < nchunks)
                def _():
                    issue_idx(n + 2, (b + 2) % DEPTH)

    @pl.when(nchunks == MAX_CHUNKS)
    def _():
        wait_scatter((MAX_CHUNKS - 1) % DEPTH)

    @pl.when(nchunks == MAX_CHUNKS - 1)
    def _():
        wait_scatter((MAX_CHUNKS - 2) % DEPTH)

    plsc.subcore_barrier()

    def _writeout(j, _):
        row0 = row_base + j * WO_ROWS
        pltpu.sync_copy(acc_sh.at[pl.ds(row0, WO_ROWS)],
                        out_hbm.at[c, pl.ds(row0, WO_ROWS)])
        return _

    lax.fori_loop(0, wo_trips, _writeout, None)


_cp = pltpu.CompilerParams()
if "needs_layout_passes" in pltpu.CompilerParams.__dataclass_fields__:
    _cp = dataclasses.replace(_cp, needs_layout_passes=False)

_sc_call = functools.partial(
    pl.kernel,
    compiler_params=_cp,
    out_type=jax.ShapeDtypeStruct((NC, N_NODES, D_FEAT), jnp.float32),
    mesh=plsc.VectorSubcoreMesh(core_axis_name="c", subcore_axis_name="s"),
    scratch_types=[
        pltpu.VMEM((DEPTH, 2, CHUNK), jnp.int32),
        pltpu.VMEM((DEPTH, CHUNK), jnp.float32),
        pltpu.VMEM((DEPTH, CHUNK, D_FEAT), jnp.float32),
        pltpu.VMEM_SHARED((N_NODES, D_FEAT), jnp.float32),
    ] + [pltpu.SemaphoreType.DMA] * (4 * DEPTH),
)(_sc_body)


def _merge_body(p_ref, o_ref):
    o_ref[...] = p_ref[0] + p_ref[1]


MERGE_BLK = 2000

_merge = pl.pallas_call(
    _merge_body,
    grid=(N_NODES // MERGE_BLK,),
    in_specs=[pl.BlockSpec((NC, MERGE_BLK, D_FEAT), lambda i: (0, i, 0))],
    out_specs=pl.BlockSpec((MERGE_BLK, D_FEAT), lambda i: (i, 0)),
    out_shape=jax.ShapeDtypeStruct((N_NODES, D_FEAT), jnp.float32),
)


@jax.jit
def kernel(h, edge_index, edge_weight):
    wr = edge_weight.reshape(1, N_EDGES)
    partials = _sc_call(h, edge_index.astype(jnp.int32), wr)
    return _merge(partials)


# speedup vs baseline: 13.4635x; 1.0134x over previous
import dataclasses
import functools

import jax
import jax.numpy as jnp
from jax import lax
from jax.experimental import pallas as pl
from jax.experimental.pallas import tpu as pltpu
from jax.experimental.pallas import tpu_sc as plsc

N_NODES = 10000
N_EDGES = 320000
D_FEAT = 128

NC = 2
NS = 16
NW = NC * NS
CHUNK = 128
TOTAL_CHUNKS = N_EDGES // CHUNK
MAX_CHUNKS = -(-TOTAL_CHUNKS // NW)
LANES = 16
DEPTH = 3
WO_ROWS = 80


def _sc_body(h_hbm, ei_hbm, w_hbm, out_hbm, idx_v, w_v, rows_v, acc_sh,
             *sems):
    isem = sems[0:DEPTH]
    gsem = sems[DEPTH:3 * DEPTH]
    ssem = sems[3 * DEPTH:4 * DEPTH]

    c = lax.axis_index("c")
    s = lax.axis_index("s")
    wid = c * NS + s
    nchunks = jnp.where(wid < TOTAL_CHUNKS - (MAX_CHUNKS - 1) * NW,
                        MAX_CHUNKS, MAX_CHUNKS - 1)

    def issue_idx(n, j):
        off = (wid + n * NW) * CHUNK
        pltpu.async_copy(ei_hbm.at[:, pl.ds(off, CHUNK)],
                         idx_v.at[j], isem[j])
        pltpu.async_copy(w_hbm.at[0, pl.ds(off, CHUNK)], w_v.at[j], isem[j])

    def wait_idx(j):
        pltpu.make_async_copy(ei_hbm.at[:, pl.ds(0, CHUNK)], idx_v.at[j],
                              isem[j]).wait()
        pltpu.make_async_copy(w_hbm.at[0, pl.ds(0, CHUNK)], w_v.at[j],
                              isem[j]).wait()

    HALF = CHUNK // 2

    def issue_gather(j):
        for hh in range(2):
            pltpu.async_copy(
                h_hbm.at[idx_v.at[j, 0, pl.ds(hh * HALF, HALF)]],
                rows_v.at[j, pl.ds(hh * HALF, HALF)], gsem[2 * j + hh])

    def wait_gather_half(j, hh):
        pltpu.make_async_copy(
            h_hbm.at[idx_v.at[j, 0, pl.ds(hh * HALF, HALF)]],
            rows_v.at[j, pl.ds(hh * HALF, HALF)], gsem[2 * j + hh]).wait()

    def issue_scatter(j):
        pltpu.async_copy(rows_v.at[j], acc_sh.at[idx_v.at[j, 1]], ssem[j],
                         add=True)

    def wait_scatter(j):
        pltpu.make_async_copy(rows_v.at[j], acc_sh.at[idx_v.at[j, 1]],
                              ssem[j]).wait()

    issue_idx(0, 0)
    issue_idx(1, 1)
    wait_idx(0)
    issue_gather(0)

    row_base = s * 640
    wo_trips = jnp.where(s == NS - 1, 5, 8)

    @pl.loop(0, WO_ROWS)
    def _zero_stage(i):
        for k in range(D_FEAT // LANES):
            rows_v[0, i, pl.ds(k * LANES, LANES)] = jnp.zeros((LANES,),
                                                              jnp.float32)

    def _zero_acc(j, _):
        pltpu.sync_copy(rows_v.at[0, pl.ds(0, WO_ROWS)],
                        acc_sh.at[pl.ds(row_base + j * WO_ROWS, WO_ROWS)])
        return _

    lax.fori_loop(0, wo_trips, _zero_acc, None)

    plsc.subcore_barrier()

    @pl.loop(0, (MAX_CHUNKS + DEPTH - 1) // DEPTH)
    def _triple(m):
        for b in range(DEPTH):
            n = m * DEPTH + b

            @pl.when(n < nchunks)
            def _():
                @pl.when(n + 1 < nchunks)
                def _():
                    wait_idx((b + 1) % DEPTH)
                    issue_gather((b + 1) % DEPTH)

                for hh in range(2):
                    wait_gather_half(b, hh)

                    @pl.loop(hh * (HALF // LANES),
                             (hh + 1) * (HALF // LANES))
                    def _scale(g):
                        wv = w_v[b, pl.ds(g * LANES, LANES)]
                        for jj in range(LANES):
                            w = wv[jj]
                            i = g * LANES + jj
                            for k in range(D_FEAT // LANES):
                                sl = pl.ds(k * LANES, LANES)
                                rows_v[b, i, sl] = rows_v[b, i, sl] * w

                issue_scatter(b)

                @pl.when(n >= 1)
                def _():
                    wait_scatter((b + 2) % DEPTH)

                @pl.when(n + 2 < nchunks)
                def _():
                    issue_idx(n + 2, (b + 2) % DEPTH)

    @pl.when(nchunks == MAX_CHUNKS)
    def _():
        wait_scatter((MAX_CHUNKS - 1) % DEPTH)

    @pl.when(nchunks == MAX_CHUNKS - 1)
    def _():
        wait_scatter((MAX_CHUNKS - 2) % DEPTH)

    plsc.subcore_barrier()

    def _writeout(j, _):
        row0 = row_base + j * WO_ROWS
        pltpu.sync_copy(acc_sh.at[pl.ds(row0, WO_ROWS)],
                        out_hbm.at[c, pl.ds(row0, WO_ROWS)])
        return _

    lax.fori_loop(0, wo_trips, _writeout, None)


_cp = pltpu.CompilerParams()
if "needs_layout_passes" in pltpu.CompilerParams.__dataclass_fields__:
    _cp = dataclasses.replace(_cp, needs_layout_passes=False)

_sc_call = functools.partial(
    pl.kernel,
    compiler_params=_cp,
    out_type=jax.ShapeDtypeStruct((NC, N_NODES, D_FEAT), jnp.float32),
    mesh=plsc.VectorSubcoreMesh(core_axis_name="c", subcore_axis_name="s"),
    scratch_types=[
        pltpu.VMEM((DEPTH, 2, CHUNK), jnp.int32),
        pltpu.VMEM((DEPTH, CHUNK), jnp.float32),
        pltpu.VMEM((DEPTH, CHUNK, D_FEAT), jnp.float32),
        pltpu.VMEM_SHARED((N_NODES, D_FEAT), jnp.float32),
    ] + [pltpu.SemaphoreType.DMA] * (4 * DEPTH),
)(_sc_body)


def _merge_body(p_ref, o_ref):
    o_ref[...] = p_ref[0] + p_ref[1]


MERGE_BLK = 2000

_merge = pl.pallas_call(
    _merge_body,
    grid=(N_NODES // MERGE_BLK,),
    in_specs=[pl.BlockSpec((NC, MERGE_BLK, D_FEAT), lambda i: (0, i, 0))],
    out_specs=pl.BlockSpec((MERGE_BLK, D_FEAT), lambda i: (i, 0)),
    out_shape=jax.ShapeDtypeStruct((N_NODES, D_FEAT), jnp.float32),
)


@jax.jit
def kernel(h, edge_index, edge_weight):
    wr = edge_weight.reshape(1, N_EDGES)
    partials = _sc_call(h, edge_index.astype(jnp.int32), wr)
    return _merge(partials)
